# baseline (speedup 1.0000x reference)
import functools

import jax
import jax.numpy as jnp
from jax import lax
from jax.experimental import pallas as pl
from jax.experimental.pallas import tpu as pltpu
from jax.experimental.pallas import tpu_sc as plsc

_N = 19
_E = 342
_LAT = 512
_HID = 2048
_T = 4096
_APAD = 384
_EPAD = 384
_IROWS = 4
_ICOLS = 96
_W = 128


def _build_m(edge_idx):
    mesh = plsc.VectorSubcoreMesh(core_axis_name="c", subcore_axis_name="s")

    @functools.partial(
        pl.kernel,
        mesh=mesh,
        out_type=jax.ShapeDtypeStruct((_APAD, _W), jnp.float32),
        scratch_types=[
            pltpu.VMEM((_EPAD,), jnp.int32),
            pltpu.VMEM((_EPAD,), jnp.int32),
            pltpu.VMEM((_ICOLS,), jnp.int32),
            pltpu.VMEM((_ICOLS,), jnp.int32),
            pltpu.VMEM((_ICOLS,), jnp.int32),
            pltpu.VMEM((_ICOLS,), jnp.int32),
            pltpu.VMEM((_ICOLS, _W), jnp.float32),
            pltpu.VMEM_SHARED((_APAD, _W), jnp.float32),
        ],
    )
    def sc_count(src_hbm, dst_hbm, zeros_hbm, ones_hbm, out_hbm,
                 src_v, dst_v, idx0, idx1, idx2, idx3, ones_v, a_sh):
        wid = lax.axis_index("s") * 2 + lax.axis_index("c")

        @pl.when(wid == 0)
        def _():
            pltpu.sync_copy(src_hbm, src_v.at[pl.ds(0, _E)])
            pltpu.sync_copy(dst_hbm, dst_v.at[pl.ds(0, _E)])
            pltpu.sync_copy(zeros_hbm, a_sh)
            pltpu.sync_copy(ones_hbm, ones_v)
            lane = lax.iota(jnp.int32, 16)
            for j, idx_v in enumerate((idx0, idx1, idx2, idx3)):
                for t in range(_ICOLS // 16):
                    base = j * _ICOLS + t * 16
                    s = src_v[pl.ds(base, 16)]
                    d = dst_v[pl.ds(base, 16)]
                    valid = (base + lane) < _E
                    idx_v[pl.ds(t * 16, 16)] = jnp.where(
                        valid, d * _N + s, _N * _N)
            for idx_v in (idx0, idx1, idx2, idx3):
                pltpu.sync_copy(ones_v, a_sh.at[idx_v], add=True)
            pltpu.sync_copy(a_sh, out_hbm)

    counts = sc_count(edge_idx[0], edge_idx[1],
                      jnp.zeros((_APAD, _W), jnp.float32),
                      jnp.ones((_ICOLS, _W), jnp.float32))
    a = counts[:_N * _N, 0].reshape(_N, _N)
    return a + jnp.eye(_N, dtype=jnp.float32)


def _agg(m, x):
    return lax.dot_general(m, x, (((1,), (0,)), ((), ())),
                           precision=lax.Precision.HIGHEST,
                           preferred_element_type=jnp.float32)


def _dot(a, b):
    return jnp.dot(a, b, preferred_element_type=jnp.float32)


_P1, _P2, _P3, _P4, _END = 1, 5, 13, 21, 40
_NB1, _NB2, _NB3 = 4, 8, 8
_C1, _C2, _C3 = _HID // _NB1, _T // _NB2, _T // _NB3


def _mega_body(m_ref, x_ref, w1a_ref, b1a_ref, w1b_ref, b1b_ref,
               w2a_a, w2a_b, b2a_ref, w2b_a, w2b_b, b2b_ref,
               wc1_a, wc1_b, wc1_c, wc1_d, bc1_ref,
               wc2_ref, bc2_ref, pred_ref, gin_ref,
               h1_s, g_s, x2_s, h2_s, gin_s, row_s, acc_s):
    i = pl.program_id(0)

    @pl.when(i == 0)
    def _():
        hin = _agg(m_ref[...], x_ref[...])
        h1_s[...] = jnp.maximum(_dot(hin, w1a_ref[...]) + b1a_ref[...], 0.0)

    @pl.when((i >= _P1) & (i < _P2))
    def _():
        blk = _dot(h1_s[...], w1b_ref[...]) + b1b_ref[...]
        for t in range(_NB1):
            @pl.when(i == _P1 + t)
            def _(t=t, blk=blk):
                g_s[:, t * _C1:(t + 1) * _C1] = blk

    @pl.when(i == _P2)
    def _():
        x2_s[...] = _agg(m_ref[...], jnp.maximum(g_s[...], 0.0))

    @pl.when((i >= _P2) & (i < _P3))
    def _():
        blk = jnp.maximum(jnp.concatenate(
            [_dot(x2_s[...], w2a_a[...]), _dot(x2_s[...], w2a_b[...])],
            axis=1) + b2a_ref[...], 0.0)
        for t in range(_NB2):
            @pl.when(i == _P2 + t)
            def _(t=t, blk=blk):
                h2_s[:, t * _C2:(t + 1) * _C2] = blk

    @pl.when((i >= _P3) & (i < _P4))
    def _():
        blk = jnp.concatenate(
            [_dot(h2_s[...], w2b_a[...]), _dot(h2_s[...], w2b_b[...])],
            axis=1) + b2b_ref[...]
        gin_ref[...] = blk
        for t in range(_NB3):
            @pl.when(i == _P3 + t)
            def _(t=t, blk=blk):
                gin_s[:, t * _C3:(t + 1) * _C3] = blk

    @pl.when(i >= _P4)
    def _():
        for k in range(_N):
            @pl.when(i == _P4 + k)
            def _(k=k):
                row_s[...] = gin_s[k:k + 1, :]
        part = jnp.concatenate(
            [_dot(row_s[...], wc1_a[...]), _dot(row_s[...], wc1_b[...]),
             _dot(row_s[...], wc1_c[...]), _dot(row_s[...], wc1_d[...])],
            axis=1)

        @pl.when(i == _P4)
        def _():
            acc_s[...] = part

        @pl.when(i > _P4)
        def _():
            acc_s[...] += part

        @pl.when(i == _END - 1)
        def _():
            z = acc_s[...] + bc1_ref[...]
            p = lax.dot_general(z, wc2_ref[...], (((1,), (0,)), ((), ())),
                                precision=lax.Precision.HIGHEST,
                                preferred_element_type=jnp.float32)
            pred_ref[...] = jax.nn.sigmoid(p + bc2_ref[...])


_ARB = pltpu.CompilerParams(dimension_semantics=("arbitrary",))


def kernel(latent_z, edge_idx, W1a, b1a, W1b, b1b, W2a, b2a, W2b, b2b,
           Wc1, bc1, Wc2, bc2):
    m = _build_m(edge_idx)

    pred, gin = pl.pallas_call(
        _mega_body,
        grid=(_END,),
        in_specs=[
            pl.BlockSpec((_N, _N), lambda i: (0, 0)),
            pl.BlockSpec((_N, _LAT), lambda i: (0, 0)),
            pl.BlockSpec((_LAT, _HID), lambda i: (0, 0)),
            pl.BlockSpec((1, _HID), lambda i: (0, 0)),
            pl.BlockSpec((_HID, _C1), lambda i: (0, jnp.clip(i - _P1, 0, _NB1 - 1))),
            pl.BlockSpec((1, _C1), lambda i: (0, jnp.clip(i - _P1, 0, _NB1 - 1))),
            pl.BlockSpec((_HID, _C2 // 2), lambda i: (0, 2 * jnp.clip(i - _P2, 0, _NB2 - 1))),
            pl.BlockSpec((_HID, _C2 // 2), lambda i: (0, 2 * jnp.clip(i - _P2, 0, _NB2 - 1) + 1)),
            pl.BlockSpec((1, _C2), lambda i: (0, jnp.clip(i - _P2, 0, _NB2 - 1))),
            pl.BlockSpec((_T, _C3 // 2), lambda i: (0, 2 * jnp.clip(i - _P3, 0, _NB3 - 1))),
            pl.BlockSpec((_T, _C3 // 2), lambda i: (0, 2 * jnp.clip(i - _P3, 0, _NB3 - 1) + 1)),
            pl.BlockSpec((1, _C3), lambda i: (0, jnp.clip(i - _P3, 0, _NB3 - 1))),
            pl.BlockSpec((_T, _LAT // 4), lambda i: (jnp.clip(i - _P4, 0, _N - 1), 0)),
            pl.BlockSpec((_T, _LAT // 4), lambda i: (jnp.clip(i - _P4, 0, _N - 1), 1)),
            pl.BlockSpec((_T, _LAT // 4), lambda i: (jnp.clip(i - _P4, 0, _N - 1), 2)),
            pl.BlockSpec((_T, _LAT // 4), lambda i: (jnp.clip(i - _P4, 0, _N - 1), 3)),
            pl.BlockSpec((1, _LAT), lambda i: (0, 0)),
            pl.BlockSpec((_LAT, 1), lambda i: (0, 0)),
            pl.BlockSpec((1, 1), lambda i: (0, 0)),
        ],
        out_specs=[
            pl.BlockSpec((1, 1), lambda i: (0, 0)),
            pl.BlockSpec((_N, _C3), lambda i: (0, jnp.clip(i - _P3, 0, _NB3 - 1))),
        ],
        out_shape=[
            jax.ShapeDtypeStruct((1, 1), jnp.float32),
            jax.ShapeDtypeStruct((_N, _T), jnp.float32),
        ],
        scratch_shapes=[
            pltpu.VMEM((_N, _HID), jnp.float32),
            pltpu.VMEM((_N, _HID), jnp.float32),
            pltpu.VMEM((_N, _HID), jnp.float32),
            pltpu.VMEM((_N, _T), jnp.float32),
            pltpu.VMEM((_N, _T), jnp.float32),
            pltpu.VMEM((1, _T), jnp.float32),
            pltpu.VMEM((1, _LAT), jnp.float32),
        ],
        compiler_params=_ARB,
    )(m, latent_z, W1a, b1a.reshape(1, -1), W1b, b1b.reshape(1, -1),
      W2a, W2a, b2a.reshape(1, -1), W2b, W2b, b2b.reshape(1, -1),
      Wc1, Wc1, Wc1, Wc1, bc1.reshape(1, -1), Wc2, bc2.reshape(1, -1))

    return (pred.reshape(1), gin)


# speedup vs baseline: 1.0164x; 1.0164x over previous
import functools

import jax
import jax.numpy as jnp
from jax import lax
from jax.experimental import pallas as pl
from jax.experimental.pallas import tpu as pltpu
from jax.experimental.pallas import tpu_sc as plsc

_N = 19
_E = 342
_LAT = 512
_HID = 2048
_T = 4096
_APAD = 384
_EPAD = 384
_IROWS = 4
_ICOLS = 96
_W = 128


def _build_m(edge_idx):
    mesh = plsc.VectorSubcoreMesh(core_axis_name="c", subcore_axis_name="s")

    @functools.partial(
        pl.kernel,
        mesh=mesh,
        out_type=jax.ShapeDtypeStruct((_APAD, _W), jnp.float32),
        scratch_types=[
            pltpu.VMEM((_EPAD,), jnp.int32),
            pltpu.VMEM((_EPAD,), jnp.int32),
            pltpu.VMEM((_ICOLS,), jnp.int32),
            pltpu.VMEM((_ICOLS,), jnp.int32),
            pltpu.VMEM((_ICOLS,), jnp.int32),
            pltpu.VMEM((_ICOLS,), jnp.int32),
            pltpu.VMEM((_ICOLS, _W), jnp.float32),
            pltpu.VMEM_SHARED((_APAD, _W), jnp.float32),
            pltpu.SemaphoreType.DMA,
        ],
    )
    def sc_count(src_hbm, dst_hbm, zeros_hbm, ones_hbm, out_hbm,
                 src_v, dst_v, idx0, idx1, idx2, idx3, ones_v, a_sh, sem):
        wid = lax.axis_index("s") * 2 + lax.axis_index("c")

        @pl.when(wid == 0)
        def _():
            cps = [pltpu.async_copy(src_hbm, src_v.at[pl.ds(0, _E)], sem),
                   pltpu.async_copy(dst_hbm, dst_v.at[pl.ds(0, _E)], sem),
                   pltpu.async_copy(zeros_hbm, a_sh, sem),
                   pltpu.async_copy(ones_hbm, ones_v, sem)]
            for c in cps:
                c.wait()
            lane = lax.iota(jnp.int32, 16)
            for j, idx_v in enumerate((idx0, idx1, idx2, idx3)):
                for t in range(_ICOLS // 16):
                    base = j * _ICOLS + t * 16
                    s = src_v[pl.ds(base, 16)]
                    d = dst_v[pl.ds(base, 16)]
                    valid = (base + lane) < _E
                    idx_v[pl.ds(t * 16, 16)] = jnp.where(
                        valid, d * _N + s, _N * _N)
            scs = [pltpu.async_copy(ones_v, a_sh.at[idx_v], sem, add=True)
                   for idx_v in (idx0, idx1, idx2, idx3)]
            for c in scs:
                c.wait()
            pltpu.sync_copy(a_sh, out_hbm)

    counts = sc_count(edge_idx[0], edge_idx[1],
                      jnp.zeros((_APAD, _W), jnp.float32),
                      jnp.ones((_ICOLS, _W), jnp.float32))
    a = counts[:_N * _N, 0].reshape(_N, _N)
    return a + jnp.eye(_N, dtype=jnp.float32)


def _agg(m, x):
    return lax.dot_general(m, x, (((1,), (0,)), ((), ())),
                           precision=lax.Precision.HIGHEST,
                           preferred_element_type=jnp.float32)


def _dot(a, b):
    return jnp.dot(a, b, preferred_element_type=jnp.float32)


_P1, _P2, _P3, _P4, _END = 1, 5, 13, 21, 40
_NB1, _NB2, _NB3 = 4, 8, 8
_C1, _C2, _C3 = _HID // _NB1, _T // _NB2, _T // _NB3


def _mega_body(m_ref, x_ref, w1a_ref, b1a_ref, w1b_ref, b1b_ref,
               w2a_a, w2a_b, b2a_ref, w2b_a, w2b_b, b2b_ref,
               wc1_a, wc1_b, wc1_c, wc1_d, bc1_ref,
               wc2_ref, bc2_ref, pred_ref, gin_ref,
               h1_s, g_s, x2_s, h2_s, gin_s, row_s, acc_s):
    i = pl.program_id(0)

    @pl.when(i == 0)
    def _():
        hin = _agg(m_ref[...], x_ref[...])
        h1_s[...] = jnp.maximum(_dot(hin, w1a_ref[...]) + b1a_ref[...], 0.0)

    @pl.when((i >= _P1) & (i < _P2))
    def _():
        blk = _dot(h1_s[...], w1b_ref[...]) + b1b_ref[...]
        for t in range(_NB1):
            @pl.when(i == _P1 + t)
            def _(t=t, blk=blk):
                g_s[:, t * _C1:(t + 1) * _C1] = blk

    @pl.when(i == _P2)
    def _():
        x2_s[...] = _agg(m_ref[...], jnp.maximum(g_s[...], 0.0))

    @pl.when((i >= _P2) & (i < _P3))
    def _():
        blk = jnp.maximum(jnp.concatenate(
            [_dot(x2_s[...], w2a_a[...]), _dot(x2_s[...], w2a_b[...])],
            axis=1) + b2a_ref[...], 0.0)
        for t in range(_NB2):
            @pl.when(i == _P2 + t)
            def _(t=t, blk=blk):
                h2_s[:, t * _C2:(t + 1) * _C2] = blk

    @pl.when((i >= _P3) & (i < _P4))
    def _():
        blk = jnp.concatenate(
            [_dot(h2_s[...], w2b_a[...]), _dot(h2_s[...], w2b_b[...])],
            axis=1) + b2b_ref[...]
        gin_ref[...] = blk
        for t in range(_NB3):
            @pl.when(i == _P3 + t)
            def _(t=t, blk=blk):
                gin_s[:, t * _C3:(t + 1) * _C3] = blk

    @pl.when(i >= _P4)
    def _():
        for k in range(_N):
            @pl.when(i == _P4 + k)
            def _(k=k):
                row_s[...] = gin_s[k:k + 1, :]
        part = jnp.concatenate(
            [_dot(row_s[...], wc1_a[...]), _dot(row_s[...], wc1_b[...]),
             _dot(row_s[...], wc1_c[...]), _dot(row_s[...], wc1_d[...])],
            axis=1)

        @pl.when(i == _P4)
        def _():
            acc_s[...] = part

        @pl.when(i > _P4)
        def _():
            acc_s[...] += part

        @pl.when(i == _END - 1)
        def _():
            z = acc_s[...] + bc1_ref[...]
            p = lax.dot_general(z, wc2_ref[...], (((1,), (0,)), ((), ())),
                                precision=lax.Precision.HIGHEST,
                                preferred_element_type=jnp.float32)
            pred_ref[...] = jax.nn.sigmoid(p + bc2_ref[...])


_ARB = pltpu.CompilerParams(dimension_semantics=("arbitrary",))


def kernel(latent_z, edge_idx, W1a, b1a, W1b, b1b, W2a, b2a, W2b, b2b,
           Wc1, bc1, Wc2, bc2):
    m = _build_m(edge_idx)

    pred, gin = pl.pallas_call(
        _mega_body,
        grid=(_END,),
        in_specs=[
            pl.BlockSpec((_N, _N), lambda i: (0, 0)),
            pl.BlockSpec((_N, _LAT), lambda i: (0, 0)),
            pl.BlockSpec((_LAT, _HID), lambda i: (0, 0)),
            pl.BlockSpec((1, _HID), lambda i: (0, 0)),
            pl.BlockSpec((_HID, _C1), lambda i: (0, jnp.clip(i - _P1, 0, _NB1 - 1))),
            pl.BlockSpec((1, _C1), lambda i: (0, jnp.clip(i - _P1, 0, _NB1 - 1))),
            pl.BlockSpec((_HID, _C2 // 2), lambda i: (0, 2 * jnp.clip(i - _P2, 0, _NB2 - 1))),
            pl.BlockSpec((_HID, _C2 // 2), lambda i: (0, 2 * jnp.clip(i - _P2, 0, _NB2 - 1) + 1)),
            pl.BlockSpec((1, _C2), lambda i: (0, jnp.clip(i - _P2, 0, _NB2 - 1))),
            pl.BlockSpec((_T, _C3 // 2), lambda i: (0, 2 * jnp.clip(i - _P3, 0, _NB3 - 1))),
            pl.BlockSpec((_T, _C3 // 2), lambda i: (0, 2 * jnp.clip(i - _P3, 0, _NB3 - 1) + 1)),
            pl.BlockSpec((1, _C3), lambda i: (0, jnp.clip(i - _P3, 0, _NB3 - 1))),
            pl.BlockSpec((_T, _LAT // 4), lambda i: (jnp.clip(i - _P4, 0, _N - 1), 0)),
            pl.BlockSpec((_T, _LAT // 4), lambda i: (jnp.clip(i - _P4, 0, _N - 1), 1)),
            pl.BlockSpec((_T, _LAT // 4), lambda i: (jnp.clip(i - _P4, 0, _N - 1), 2)),
            pl.BlockSpec((_T, _LAT // 4), lambda i: (jnp.clip(i - _P4, 0, _N - 1), 3)),
            pl.BlockSpec((1, _LAT), lambda i: (0, 0)),
            pl.BlockSpec((_LAT, 1), lambda i: (0, 0)),
            pl.BlockSpec((1, 1), lambda i: (0, 0)),
        ],
        out_specs=[
            pl.BlockSpec((1, 1), lambda i: (0, 0)),
            pl.BlockSpec((_N, _C3), lambda i: (0, jnp.clip(i - _P3, 0, _NB3 - 1))),
        ],
        out_shape=[
            jax.ShapeDtypeStruct((1, 1), jnp.float32),
            jax.ShapeDtypeStruct((_N, _T), jnp.float32),
        ],
        scratch_shapes=[
            pltpu.VMEM((_N, _HID), jnp.float32),
            pltpu.VMEM((_N, _HID), jnp.float32),
            pltpu.VMEM((_N, _HID), jnp.float32),
            pltpu.VMEM((_N, _T), jnp.float32),
            pltpu.VMEM((_N, _T), jnp.float32),
            pltpu.VMEM((1, _T), jnp.float32),
            pltpu.VMEM((1, _LAT), jnp.float32),
        ],
        compiler_params=_ARB,
    )(m, latent_z, W1a, b1a.reshape(1, -1), W1b, b1b.reshape(1, -1),
      W2a, W2a, b2a.reshape(1, -1), W2b, W2b, b2b.reshape(1, -1),
      Wc1, Wc1, Wc1, Wc1, bc1.reshape(1, -1), Wc2, bc2.reshape(1, -1))

    return (pred.reshape(1), gin)


# speedup vs baseline: 1.0214x; 1.0049x over previous
import functools

import jax
import jax.numpy as jnp
from jax import lax
from jax.experimental import pallas as pl
from jax.experimental.pallas import tpu as pltpu
from jax.experimental.pallas import tpu_sc as plsc

_N = 19
_E = 342
_LAT = 512
_HID = 2048
_T = 4096
_APAD = 384
_EPAD = 384
_ICOLS = 96
_W = 128


def _build_m(edge_idx):
    mesh = plsc.VectorSubcoreMesh(core_axis_name="c", subcore_axis_name="s")

    @functools.partial(
        pl.kernel,
        mesh=mesh,
        out_type=jax.ShapeDtypeStruct((_APAD, _W), jnp.float32),
        scratch_types=[
            pltpu.VMEM((_EPAD,), jnp.int32),
            pltpu.VMEM((_EPAD,), jnp.int32),
            pltpu.VMEM((_ICOLS,), jnp.int32),
            pltpu.VMEM((_ICOLS,), jnp.int32),
            pltpu.VMEM((_ICOLS,), jnp.int32),
            pltpu.VMEM((_ICOLS,), jnp.int32),
            pltpu.VMEM((_ICOLS, _W), jnp.float32),
            pltpu.VMEM_SHARED((_APAD, _W), jnp.float32),
            pltpu.SemaphoreType.DMA,
        ],
    )
    def sc_count(src_hbm, dst_hbm, zeros_hbm, ones_hbm, out_hbm,
                 src_v, dst_v, idx0, idx1, idx2, idx3, ones_v, a_sh, sem):
        wid = lax.axis_index("s") * 2 + lax.axis_index("c")

        @pl.when(wid == 0)
        def _():
            cps = [pltpu.async_copy(src_hbm, src_v.at[pl.ds(0, _E)], sem),
                   pltpu.async_copy(dst_hbm, dst_v.at[pl.ds(0, _E)], sem),
                   pltpu.async_copy(zeros_hbm, a_sh, sem),
                   pltpu.async_copy(ones_hbm, ones_v, sem)]
            for c in cps:
                c.wait()
            lane = lax.iota(jnp.int32, 16)
            for j, idx_v in enumerate((idx0, idx1, idx2, idx3)):
                for t in range(_ICOLS // 16):
                    base = j * _ICOLS + t * 16
                    s = src_v[pl.ds(base, 16)]
                    d = dst_v[pl.ds(base, 16)]
                    valid = (base + lane) < _E
                    idx_v[pl.ds(t * 16, 16)] = jnp.where(
                        valid, d * _N + s, _N * _N)
            scs = [pltpu.async_copy(ones_v, a_sh.at[idx_v], sem, add=True)
                   for idx_v in (idx0, idx1, idx2, idx3)]
            for c in scs:
                c.wait()
            pltpu.sync_copy(a_sh, out_hbm)

    counts = sc_count(edge_idx[0], edge_idx[1],
                      jnp.zeros((_APAD, _W), jnp.float32),
                      jnp.ones((_ICOLS, _W), jnp.float32))
    a = counts[:_N * _N, 0].reshape(_N, _N)
    return a + jnp.eye(_N, dtype=jnp.float32)


def _agg(m, x):
    return lax.dot_general(m, x, (((1,), (0,)), ((), ())),
                           precision=lax.Precision.HIGHEST,
                           preferred_element_type=jnp.float32)


def _dot(a, b):
    return jnp.dot(a, b, preferred_element_type=jnp.float32)


_P1, _P2, _P3, _P4, _END = 1, 5, 13, 21, 40
_NB1, _NB2, _NB3 = 4, 8, 8
_C1, _C2, _C3 = _HID // _NB1, _T // _NB2, _T // _NB3


def _mega_body(m_ref, x_ref, w1a_ref, b1a_ref, w1b_ref, b1b_ref,
               w2a_a, w2a_b, b2a_ref, w2b_a, w2b_b, b2b_ref,
               wc1_a, wc1_b, wc1_c, wc1_d, bc1_ref,
               wc2_ref, bc2_ref, pred_ref, gin_ref,
               h1_s, g_s, x2_s, h2_s, gin_s, row_s, acc_s):
    i = pl.program_id(0)

    @pl.when(i == 0)
    def _():
        hin = _agg(m_ref[...], x_ref[...])
        h1_s[...] = jnp.maximum(_dot(hin, w1a_ref[...]) + b1a_ref[...], 0.0)

    @pl.when((i >= _P1) & (i < _P2))
    def _():
        blk = _dot(h1_s[...], w1b_ref[...]) + b1b_ref[...]
        for t in range(_NB1):
            @pl.when(i == _P1 + t)
            def _(t=t, blk=blk):
                g_s[:, t * _C1:(t + 1) * _C1] = blk

    @pl.when(i == _P2)
    def _():
        x2_s[...] = _agg(m_ref[...], jnp.maximum(g_s[...], 0.0))

    @pl.when((i >= _P2) & (i < _P3))
    def _():
        blk = jnp.maximum(jnp.concatenate(
            [_dot(x2_s[...], w2a_a[...]), _dot(x2_s[...], w2a_b[...])],
            axis=1) + b2a_ref[...], 0.0)
        for t in range(_NB2):
            @pl.when(i == _P2 + t)
            def _(t=t, blk=blk):
                h2_s[:, t * _C2:(t + 1) * _C2] = blk

    @pl.when((i >= _P3) & (i < _P4))
    def _():
        blk = jnp.concatenate(
            [_dot(h2_s[...], w2b_a[...]), _dot(h2_s[...], w2b_b[...])],
            axis=1) + b2b_ref[...]
        gin_ref[...] = blk
        for t in range(_NB3):
            @pl.when(i == _P3 + t)
            def _(t=t, blk=blk):
                gin_s[:, t * _C3:(t + 1) * _C3] = blk

    @pl.when(i >= _P4)
    def _():
        for k in range(_N):
            @pl.when(i == _P4 + k)
            def _(k=k):
                row_s[...] = gin_s[k:k + 1, :]
        part = jnp.concatenate(
            [_dot(row_s[...], wc1_a[...]), _dot(row_s[...], wc1_b[...]),
             _dot(row_s[...], wc1_c[...]), _dot(row_s[...], wc1_d[...])],
            axis=1)

        @pl.when(i == _P4)
        def _():
            acc_s[...] = part

        @pl.when(i > _P4)
        def _():
            acc_s[...] += part

        @pl.when(i == _END - 1)
        def _():
            z = acc_s[...] + bc1_ref[...]
            p = lax.dot_general(z, wc2_ref[...], (((1,), (0,)), ((), ())),
                                precision=lax.Precision.HIGHEST,
                                preferred_element_type=jnp.float32)
            pred_ref[...] = jax.nn.sigmoid(p + bc2_ref[...])


_ARB = pltpu.CompilerParams(dimension_semantics=("arbitrary",))


def kernel(latent_z, edge_idx, W1a, b1a, W1b, b1b, W2a, b2a, W2b, b2b,
           Wc1, bc1, Wc2, bc2):
    m = _build_m(edge_idx)

    pred, gin = pl.pallas_call(
        _mega_body,
        grid=(_END,),
        in_specs=[
            pl.BlockSpec((_N, _N), lambda i: (0, 0)),
            pl.BlockSpec((_N, _LAT), lambda i: (0, 0)),
            pl.BlockSpec((_LAT, _HID), lambda i: (0, 0)),
            pl.BlockSpec((1, _HID), lambda i: (0, 0)),
            pl.BlockSpec((_HID, _C1), lambda i: (0, jnp.clip(i - _P1, 0, _NB1 - 1))),
            pl.BlockSpec((1, _C1), lambda i: (0, jnp.clip(i - _P1, 0, _NB1 - 1))),
            pl.BlockSpec((_HID, _C2 // 2), lambda i: (0, 2 * jnp.clip(i - _P2, 0, _NB2 - 1))),
            pl.BlockSpec((_HID, _C2 // 2), lambda i: (0, 2 * jnp.clip(i - _P2, 0, _NB2 - 1) + 1)),
            pl.BlockSpec((1, _C2), lambda i: (0, jnp.clip(i - _P2, 0, _NB2 - 1))),
            pl.BlockSpec((_T, _C3 // 2), lambda i: (0, 2 * jnp.clip(i - _P3, 0, _NB3 - 1))),
            pl.BlockSpec((_T, _C3 // 2), lambda i: (0, 2 * jnp.clip(i - _P3, 0, _NB3 - 1) + 1)),
            pl.BlockSpec((1, _C3), lambda i: (0, jnp.clip(i - _P3, 0, _NB3 - 1))),
            pl.BlockSpec((_T, _LAT // 4), lambda i: (jnp.clip(i - _P4, 0, _N - 1), 0)),
            pl.BlockSpec((_T, _LAT // 4), lambda i: (jnp.clip(i - _P4, 0, _N - 1), 1)),
            pl.BlockSpec((_T, _LAT // 4), lambda i: (jnp.clip(i - _P4, 0, _N - 1), 2)),
            pl.BlockSpec((_T, _LAT // 4), lambda i: (jnp.clip(i - _P4, 0, _N - 1), 3)),
            pl.BlockSpec((1, _LAT), lambda i: (0, 0)),
            pl.BlockSpec((_LAT, 1), lambda i: (0, 0)),
            pl.BlockSpec((1, 1), lambda i: (0, 0)),
        ],
        out_specs=[
            pl.BlockSpec((1, 1), lambda i: (0, 0)),
            pl.BlockSpec((_N, _C3), lambda i: (0, jnp.clip(i - _P3, 0, _NB3 - 1))),
        ],
        out_shape=[
            jax.ShapeDtypeStruct((1, 1), jnp.float32),
            jax.ShapeDtypeStruct((_N, _T), jnp.float32),
        ],
        scratch_shapes=[
            pltpu.VMEM((_N, _HID), jnp.float32),
            pltpu.VMEM((_N, _HID), jnp.float32),
            pltpu.VMEM((_N, _HID), jnp.float32),
            pltpu.VMEM((_N, _T), jnp.float32),
            pltpu.VMEM((_N, _T), jnp.float32),
            pltpu.VMEM((1, _T), jnp.float32),
            pltpu.VMEM((1, _LAT), jnp.float32),
        ],
        compiler_params=_ARB,
    )(m, latent_z, W1a, b1a.reshape(1, -1), W1b, b1b.reshape(1, -1),
      W2a, W2a, b2a.reshape(1, -1), W2b, W2b, b2b.reshape(1, -1),
      Wc1, Wc1, Wc1, Wc1, bc1.reshape(1, -1), Wc2, bc2.reshape(1, -1))

    return (pred.reshape(1), gin)
